# baseline (speedup 1.0000x reference)
import functools

import jax
import jax.numpy as jnp
from jax import lax
from jax.experimental import pallas as pl
from jax.experimental.pallas import tpu as pltpu
from jax.experimental.pallas import tpu_sc as plsc

TILE = 192


def _row_gather_call(table, idx, n_out, d):
    info = plsc.get_sparse_core_info()
    nc, ns = info.num_cores, info.num_subcores
    nw = nc * ns
    bpw = n_out // nw
    mesh = plsc.VectorSubcoreMesh(core_axis_name="c", subcore_axis_name="s")

    @functools.partial(
        pl.kernel,
        mesh=mesh,
        out_type=jax.ShapeDtypeStruct((n_out, d), jnp.float32),
        scratch_types=[
            pltpu.VMEM((bpw,), jnp.int32),
            pltpu.VMEM((bpw, d), jnp.float32),
            pltpu.SemaphoreType.DMA,
        ],
    )
    def gather_k(table_hbm, idx_hbm, out_hbm, idx_v, rows_v, sem):
        wid = lax.axis_index("s") * nc + lax.axis_index("c")
        base = wid * bpw
        pltpu.sync_copy(idx_hbm.at[pl.ds(base, bpw)], idx_v)
        pltpu.async_copy(table_hbm.at[idx_v], rows_v, sem).wait()
        pltpu.sync_copy(rows_v, out_hbm.at[pl.ds(base, bpw)])

    return gather_k(table, idx)


def _row_scatter_call(values, idx, n_out, d):
    n_in = values.shape[0]
    info = plsc.get_sparse_core_info()
    nc, ns = info.num_cores, info.num_subcores
    nw = nc * ns
    bpw = n_in // nw
    mesh = plsc.VectorSubcoreMesh(core_axis_name="c", subcore_axis_name="s")

    @functools.partial(
        pl.kernel,
        mesh=mesh,
        out_type=jax.ShapeDtypeStruct((n_out, d), values.dtype),
        scratch_types=[
            pltpu.VMEM((bpw,), jnp.int32),
            pltpu.VMEM((bpw, d), values.dtype),
            pltpu.SemaphoreType.DMA,
        ],
    )
    def scatter_k(vals_hbm, idx_hbm, out_hbm, idx_v, rows_v, sem):
        wid = lax.axis_index("s") * nc + lax.axis_index("c")
        base = wid * bpw
        pltpu.sync_copy(idx_hbm.at[pl.ds(base, bpw)], idx_v)
        pltpu.sync_copy(vals_hbm.at[pl.ds(base, bpw)], rows_v)
        pltpu.async_copy(rows_v, out_hbm.at[idx_v], sem).wait()

    return scatter_k(values, idx)


def _matmul_tile_kernel(texp_ref, x_ref, w_ref, b_ref, o_ref):
    del texp_ref
    o_ref[...] = (
        jnp.dot(x_ref[...], w_ref[0], preferred_element_type=jnp.float32)
        + b_ref[0, 0]
    )


def kernel(inputs, index, weight, bias):
    tokens, in_f = inputs.shape
    n_exp, _, out_f = weight.shape

    import math as _math

    align = TILE * 256 // _math.gcd(TILE, 256)
    p_rows = tokens + n_exp * (TILE - 1)
    p_rows = ((p_rows + align - 1) // align) * align
    n_tiles = p_rows // TILE

    idx = index.astype(jnp.int32)
    oh = (idx[:, None] == jnp.arange(n_exp, dtype=jnp.int32)[None, :]).astype(
        jnp.int32
    )
    ccum = jnp.cumsum(oh, axis=0)
    counts = ccum[-1]
    tiles_per_e = (counts + TILE - 1) // TILE
    tile_end = jnp.cumsum(tiles_per_e)
    row_start = (tile_end - tiles_per_e) * TILE
    pos = jnp.sum(oh * (ccum - 1 + row_start[None, :]), axis=1)
    tile_id = jnp.arange(n_tiles, dtype=jnp.int32)
    tile_expert = jnp.minimum(
        jnp.sum((tile_end[None, :] <= tile_id[:, None]).astype(jnp.int32), axis=1),
        n_exp - 1,
    ).astype(jnp.int32)

    x_sorted = _row_scatter_call(inputs, pos, p_rows, in_f)

    grid_spec = pltpu.PrefetchScalarGridSpec(
        num_scalar_prefetch=1,
        grid=(n_tiles,),
        in_specs=[
            pl.BlockSpec((TILE, in_f), lambda i, texp: (i, 0)),
            pl.BlockSpec((1, in_f, out_f), lambda i, texp: (texp[i], 0, 0)),
            pl.BlockSpec((1, 1, out_f), lambda i, texp: (texp[i], 0, 0)),
        ],
        out_specs=pl.BlockSpec((TILE, out_f), lambda i, texp: (i, 0)),
    )
    y_sorted = pl.pallas_call(
        _matmul_tile_kernel,
        grid_spec=grid_spec,
        out_shape=jax.ShapeDtypeStruct((p_rows, out_f), jnp.float32),
    )(tile_expert, x_sorted, weight, bias[:, None, :])

    return _row_gather_call(y_sorted, pos, tokens, out_f)


# speedup vs baseline: 1.0499x; 1.0499x over previous
import functools

import jax
import jax.numpy as jnp
from jax import lax
from jax.experimental import pallas as pl
from jax.experimental.pallas import tpu as pltpu
from jax.experimental.pallas import tpu_sc as plsc

TILE = 256


def _row_gather_call(table, idx, n_out, d):
    info = plsc.get_sparse_core_info()
    nc, ns = info.num_cores, info.num_subcores
    nw = nc * ns
    bpw = n_out // nw
    mesh = plsc.VectorSubcoreMesh(core_axis_name="c", subcore_axis_name="s")

    @functools.partial(
        pl.kernel,
        mesh=mesh,
        out_type=jax.ShapeDtypeStruct((n_out, d), jnp.float32),
        scratch_types=[
            pltpu.VMEM((bpw,), jnp.int32),
            pltpu.VMEM((bpw, d), jnp.float32),
            pltpu.SemaphoreType.DMA,
        ],
    )
    def gather_k(table_hbm, idx_hbm, out_hbm, idx_v, rows_v, sem):
        wid = lax.axis_index("s") * nc + lax.axis_index("c")
        base = wid * bpw
        pltpu.sync_copy(idx_hbm.at[pl.ds(base, bpw)], idx_v)
        pltpu.async_copy(table_hbm.at[idx_v], rows_v, sem).wait()
        pltpu.sync_copy(rows_v, out_hbm.at[pl.ds(base, bpw)])

    return gather_k(table, idx)


def _row_scatter_call(values, idx, n_out, d):
    n_in = values.shape[0]
    info = plsc.get_sparse_core_info()
    nc, ns = info.num_cores, info.num_subcores
    nw = nc * ns
    bpw = n_in // nw
    mesh = plsc.VectorSubcoreMesh(core_axis_name="c", subcore_axis_name="s")

    @functools.partial(
        pl.kernel,
        mesh=mesh,
        out_type=jax.ShapeDtypeStruct((n_out, d), values.dtype),
        scratch_types=[
            pltpu.VMEM((bpw,), jnp.int32),
            pltpu.VMEM((bpw, d), values.dtype),
            pltpu.SemaphoreType.DMA,
        ],
    )
    def scatter_k(vals_hbm, idx_hbm, out_hbm, idx_v, rows_v, sem):
        wid = lax.axis_index("s") * nc + lax.axis_index("c")
        base = wid * bpw
        pltpu.sync_copy(idx_hbm.at[pl.ds(base, bpw)], idx_v)
        pltpu.sync_copy(vals_hbm.at[pl.ds(base, bpw)], rows_v)
        pltpu.async_copy(rows_v, out_hbm.at[idx_v], sem).wait()

    return scatter_k(values, idx)


def _matmul_tile_kernel(texp_ref, x_ref, w_ref, b_ref, o_ref):
    e = texp_ref[pl.program_id(0)]
    o_ref[...] = (
        jnp.dot(x_ref[...], w_ref[e], preferred_element_type=jnp.float32)
        + b_ref[e, 0]
    )


def kernel(inputs, index, weight, bias):
    tokens, in_f = inputs.shape
    n_exp, _, out_f = weight.shape

    import math as _math

    align = TILE * 256 // _math.gcd(TILE, 256)
    p_rows = tokens + n_exp * (TILE - 1)
    p_rows = ((p_rows + align - 1) // align) * align
    n_tiles = p_rows // TILE

    idx = index.astype(jnp.int32)
    oh = (idx[:, None] == jnp.arange(n_exp, dtype=jnp.int32)[None, :]).astype(
        jnp.int32
    )
    ccum = jnp.cumsum(oh, axis=0)
    counts = ccum[-1]
    tiles_per_e = (counts + TILE - 1) // TILE
    tile_end = jnp.cumsum(tiles_per_e)
    row_start = (tile_end - tiles_per_e) * TILE
    pos = jnp.sum(oh * (ccum - 1 + row_start[None, :]), axis=1)
    tile_id = jnp.arange(n_tiles, dtype=jnp.int32)
    tile_expert = jnp.minimum(
        jnp.sum((tile_end[None, :] <= tile_id[:, None]).astype(jnp.int32), axis=1),
        n_exp - 1,
    ).astype(jnp.int32)

    x_sorted = _row_scatter_call(inputs, pos, p_rows, in_f)

    grid_spec = pltpu.PrefetchScalarGridSpec(
        num_scalar_prefetch=1,
        grid=(n_tiles,),
        in_specs=[
            pl.BlockSpec((TILE, in_f), lambda i, texp: (i, 0)),
            pl.BlockSpec((n_exp, in_f, out_f), lambda i, texp: (0, 0, 0)),
            pl.BlockSpec((n_exp, 1, out_f), lambda i, texp: (0, 0, 0)),
        ],
        out_specs=pl.BlockSpec((TILE, out_f), lambda i, texp: (i, 0)),
    )
    y_sorted = pl.pallas_call(
        _matmul_tile_kernel,
        grid_spec=grid_spec,
        out_shape=jax.ShapeDtypeStruct((p_rows, out_f), jnp.float32),
    )(tile_expert, x_sorted, weight, bias[:, None, :])

    return _row_gather_call(y_sorted, pos, tokens, out_f)


# speedup vs baseline: 1.1326x; 1.0788x over previous
import functools

import jax
import jax.numpy as jnp
from jax import lax
from jax.experimental import pallas as pl
from jax.experimental.pallas import tpu as pltpu
from jax.experimental.pallas import tpu_sc as plsc

TILE = 256
SUB = 128


def _row_gather_call(table, idx, n_out, d):
    info = plsc.get_sparse_core_info()
    nc, ns = info.num_cores, info.num_subcores
    nw = nc * ns
    bpw = n_out // nw
    mesh = plsc.VectorSubcoreMesh(core_axis_name="c", subcore_axis_name="s")

    @functools.partial(
        pl.kernel,
        mesh=mesh,
        out_type=jax.ShapeDtypeStruct((n_out, d), jnp.float32),
        scratch_types=[
            pltpu.VMEM((bpw,), jnp.int32),
            pltpu.VMEM((bpw, d), jnp.float32),
            pltpu.SemaphoreType.DMA,
        ],
    )
    def gather_k(table_hbm, idx_hbm, out_hbm, idx_v, rows_v, sem):
        wid = lax.axis_index("s") * nc + lax.axis_index("c")
        base = wid * bpw
        pltpu.sync_copy(idx_hbm.at[pl.ds(base, bpw)], idx_v)
        pltpu.async_copy(table_hbm.at[idx_v], rows_v, sem).wait()
        pltpu.sync_copy(rows_v, out_hbm.at[pl.ds(base, bpw)])

    return gather_k(table, idx)


def _row_scatter_call(values, idx, n_out, d):
    n_in = values.shape[0]
    info = plsc.get_sparse_core_info()
    nc, ns = info.num_cores, info.num_subcores
    nw = nc * ns
    bpw = n_in // nw
    mesh = plsc.VectorSubcoreMesh(core_axis_name="c", subcore_axis_name="s")

    @functools.partial(
        pl.kernel,
        mesh=mesh,
        out_type=jax.ShapeDtypeStruct((n_out, d), values.dtype),
        scratch_types=[
            pltpu.VMEM((bpw,), jnp.int32),
            pltpu.VMEM((bpw, d), values.dtype),
            pltpu.SemaphoreType.DMA,
        ],
    )
    def scatter_k(vals_hbm, idx_hbm, out_hbm, idx_v, rows_v, sem):
        wid = lax.axis_index("s") * nc + lax.axis_index("c")
        base = wid * bpw
        pltpu.sync_copy(idx_hbm.at[pl.ds(base, bpw)], idx_v)
        pltpu.sync_copy(vals_hbm.at[pl.ds(base, bpw)], rows_v)
        pltpu.async_copy(rows_v, out_hbm.at[idx_v], sem).wait()

    return scatter_k(values, idx)


def _matmul_tile_kernel(texp_ref, x_ref, w_ref, b_ref, o_ref):
    i = pl.program_id(0)
    e0 = texp_ref[2 * i]
    e1 = texp_ref[2 * i + 1]
    o_ref[0:SUB, :] = (
        jnp.dot(x_ref[0:SUB, :], w_ref[e0], preferred_element_type=jnp.float32)
        + b_ref[e0, 0]
    )
    o_ref[SUB:TILE, :] = (
        jnp.dot(x_ref[SUB:TILE, :], w_ref[e1], preferred_element_type=jnp.float32)
        + b_ref[e1, 0]
    )


def kernel(inputs, index, weight, bias):
    tokens, in_f = inputs.shape
    n_exp, _, out_f = weight.shape

    import math as _math

    align = TILE * 256 // _math.gcd(TILE, 256)
    p_rows = tokens + n_exp * (SUB - 1)
    p_rows = ((p_rows + align - 1) // align) * align
    n_tiles = p_rows // TILE
    n_subs = p_rows // SUB

    idx = index.astype(jnp.int32)
    oh = (idx[:, None] == jnp.arange(n_exp, dtype=jnp.int32)[None, :]).astype(
        jnp.int32
    )
    ccum = jnp.cumsum(oh, axis=0)
    counts = ccum[-1]
    subs_per_e = (counts + SUB - 1) // SUB
    sub_end = jnp.cumsum(subs_per_e)
    row_start = (sub_end - subs_per_e) * SUB
    pos = jnp.sum(oh * (ccum - 1 + row_start[None, :]), axis=1)
    sub_id = jnp.arange(n_subs, dtype=jnp.int32)
    sub_expert = jnp.minimum(
        jnp.sum((sub_end[None, :] <= sub_id[:, None]).astype(jnp.int32), axis=1),
        n_exp - 1,
    ).astype(jnp.int32)

    x_sorted = _row_scatter_call(inputs, pos, p_rows, in_f)

    grid_spec = pltpu.PrefetchScalarGridSpec(
        num_scalar_prefetch=1,
        grid=(n_tiles,),
        in_specs=[
            pl.BlockSpec((TILE, in_f), lambda i, texp: (i, 0)),
            pl.BlockSpec((n_exp, in_f, out_f), lambda i, texp: (0, 0, 0)),
            pl.BlockSpec((n_exp, 1, out_f), lambda i, texp: (0, 0, 0)),
        ],
        out_specs=pl.BlockSpec((TILE, out_f), lambda i, texp: (i, 0)),
    )
    y_sorted = pl.pallas_call(
        _matmul_tile_kernel,
        grid_spec=grid_spec,
        out_shape=jax.ShapeDtypeStruct((p_rows, out_f), jnp.float32),
    )(sub_expert, x_sorted, weight, bias[:, None, :])

    return _row_gather_call(y_sorted, pos, tokens, out_f)
